# baseline (speedup 1.0000x reference)
import jax
import jax.numpy as jnp
from jax.experimental import pallas as pl
from jax.experimental.pallas import tpu as pltpu

_BEAM = 4
_VOCAB = 100000
_EOS = 2
_ROWS = 32
_LANE = 128
_NFULL = _VOCAB // _LANE
_TAILSTART = _VOCAB - _LANE
_TAILSKIP = _NFULL * _LANE - _TAILSTART
_PHASES = 4
_NEG = float("-inf")
_BIG = 2147483647


def _step(x_ref, sc_ref, eos_ref, ts_ref, wi_ref, bi_ref):
    i = pl.program_id(0)
    sc = sc_ref[i]
    eosb = eos_ref[i] != 0

    lane = jax.lax.broadcasted_iota(jnp.int32, (_ROWS, _LANE), 1)
    row8 = jax.lax.broadcasted_iota(jnp.int32, (_ROWS, _LANE), 0)
    row4 = row8 % _BEAM
    flatbase = row4 * _VOCAB + lane

    v0 = x_ref[:, 0:_LANE]
    neg = jnp.full((_ROWS, _LANE), _NEG, jnp.float32)
    zero = jnp.zeros((_ROWS, _LANE), jnp.int32)
    tv = [jnp.where(lane < 2, _NEG, v0), neg, neg, neg]
    ti = [zero, zero, zero, zero]
    for c in range(1, _NFULL):
        off = c * _LANE
        p = (c - 1) % _PHASES
        v = x_ref[:, off:off + _LANE]
        b = v > tv[p]
        tv[p] = jnp.where(b, v, tv[p])
        ti[p] = jnp.where(b, off, ti[p])
    vt = jnp.where(lane >= _TAILSKIP, x_ref[:, _TAILSTART:_TAILSTART + _LANE],
                   _NEG)
    bt = vt > tv[0]
    tv[0] = jnp.where(bt, vt, tv[0])
    ti[0] = jnp.where(bt, _TAILSTART, ti[0])

    mrow = jnp.maximum(jnp.maximum(tv[0], tv[1]), jnp.maximum(tv[2], tv[3]))
    mrow = jnp.maximum(mrow, jnp.where(lane < 2, v0, _NEG))
    m = jnp.max(mrow, axis=1, keepdims=True)

    zerof = jnp.zeros((_ROWS, _LANE), jnp.float32)
    parts = [zerof, zerof, zerof, zerof]
    for c in range(_NFULL):
        off = c * _LANE
        parts[c % _PHASES] = parts[c % _PHASES] + \
            jnp.exp(x_ref[:, off:off + _LANE] - m)
    parts[0] = parts[0] + jnp.exp(vt - m)
    ssum = (parts[0] + parts[1]) + (parts[2] + parts[3])
    s = jnp.sum(ssum, axis=1, keepdims=True)
    crow = sc - m - jnp.log(s)
    crow = jnp.where(eosb, _NEG, crow)

    cv = [t + crow for t in tv] + [jnp.where(eosb & (lane == _EOS), sc, _NEG)]
    cf = [flatbase + i for i in ti] + [flatbase]

    def _treemax(xs):
        while len(xs) > 1:
            xs = [jnp.maximum(a, b) for a, b in zip(xs[::2], xs[1::2])] + \
                ([xs[-1]] if len(xs) % 2 else [])
        return xs[0]

    def _treemin(xs):
        while len(xs) > 1:
            xs = [jnp.minimum(a, b) for a, b in zip(xs[::2], xs[1::2])] + \
                ([xs[-1]] if len(xs) % 2 else [])
        return xs[0]

    kcol = jax.lax.broadcasted_iota(jnp.int32, (1, _ROWS // _BEAM, _BEAM), 2)
    brow = jax.lax.broadcasted_iota(jnp.int32, (1, _ROWS // _BEAM, _BEAM), 1)
    ts = jnp.zeros((1, _ROWS // _BEAM, _BEAM), jnp.float32)
    wi = jnp.zeros((1, _ROWS // _BEAM, _BEAM), jnp.int32)
    bi = jnp.zeros((1, _ROWS // _BEAM, _BEAM), jnp.int32)
    v4s = []
    for b in range(_ROWS // _BEAM):
        selb = (row8 // _BEAM) == b
        av = [jnp.where(selb, v, _NEG) for v in cv]
        for k in range(_BEAM):
            mk = jnp.max(_treemax(av))
            idx = jnp.min(_treemin(
                [jnp.where(v == mk, f, _BIG) for v, f in zip(av, cf)]))
            av = [jnp.where((f == idx) & (v == mk), _NEG, v)
                  for v, f in zip(av, cf)]
            beam = idx // _VOCAB
            word = idx - beam * _VOCAB
            hit = (brow == b) & (kcol == k)
            ts = jnp.where(hit, mk, ts)
            wi = jnp.where(hit, word, wi)
            bi = jnp.where(hit, beam, bi)
            if k == _BEAM - 1:
                v4s.append(mk)

    rio = jax.lax.broadcasted_iota(jnp.int32, (_ROWS, 1), 0)
    v4b = jnp.full((_ROWS, 1), v4s[0])
    for b in range(1, _ROWS // _BEAM):
        v4b = jnp.where(rio >= b * _BEAM, v4s[b], v4b)
    cgt = [zero, zero, zero, zero]
    cge = [zero, zero, zero, zero]

    def _count(p, v):
        a = v + crow
        cgt[p] = cgt[p] + jnp.where(a > v4b, 1, 0)
        cge[p] = cge[p] + jnp.where(a >= v4b, 1, 0)

    _count(0, jnp.where(lane < 2, _NEG, v0))
    for c in range(1, _NFULL):
        off = c * _LANE
        _count((c - 1) % _PHASES, x_ref[:, off:off + _LANE])
    _count(0, vt)
    gts = (cgt[0] + cgt[1]) + (cgt[2] + cgt[3])
    ges = (cge[0] + cge[1]) + (cge[2] + cge[3])
    egt = jnp.where(eosb & (sc > v4b), 1, 0)
    ege = jnp.where(eosb & (sc >= v4b), 1, 0)
    ok = True
    for b in range(_ROWS // _BEAM):
        sel2 = (row8 // _BEAM) == b
        sel1 = (rio // _BEAM) == b
        ngt = jnp.sum(jnp.where(sel2, gts, 0)) + jnp.sum(jnp.where(sel1, egt, 0))
        nge = jnp.sum(jnp.where(sel2, ges, 0)) + jnp.sum(jnp.where(sel1, ege, 0))
        ok = ok & (ngt == 3) & (nge == 4)

    def _good():
        return ts, wi, bi

    def _bad():
        col = jax.lax.broadcasted_iota(jnp.int32, (_ROWS, _VOCAB), 1)
        rowf = jax.lax.broadcasted_iota(jnp.int32, (_ROWS, _VOCAB), 0)
        x = x_ref[...]
        adj = x + crow
        adj = jnp.where(col < 2, _NEG, adj)
        adj = jnp.where((col == _EOS) & eosb, sc, adj)
        fts = jnp.zeros((1, _ROWS // _BEAM, _BEAM), jnp.float32)
        fwi = jnp.zeros((1, _ROWS // _BEAM, _BEAM), jnp.int32)
        fbi = jnp.zeros((1, _ROWS // _BEAM, _BEAM), jnp.int32)
        for b in range(_ROWS // _BEAM):
            ab = adj[b * _BEAM:(b + 1) * _BEAM]
            fl = (rowf[b * _BEAM:(b + 1) * _BEAM] - b * _BEAM) * _VOCAB + \
                col[b * _BEAM:(b + 1) * _BEAM]
            for k in range(_BEAM):
                mk = jnp.max(ab)
                idx = jnp.min(jnp.where(ab == mk, fl, _BIG))
                ab = jnp.where(fl == idx, _NEG, ab)
                beam = idx // _VOCAB
                word = idx - beam * _VOCAB
                hit = (brow == b) & (kcol == k)
                fts = jnp.where(hit, mk, fts)
                fwi = jnp.where(hit, word, fwi)
                fbi = jnp.where(hit, beam, fbi)
        return fts, fwi, fbi

    ts, wi, bi = jax.lax.cond(ok, _good, _bad)
    ts_ref[pl.ds(i, 1)] = ts
    wi_ref[pl.ds(i, 1)] = wi
    bi_ref[pl.ds(i, 1)] = bi


def kernel(logits, scores, generated_tokens, position):
    n_rows = logits.shape[0]
    n_batch = n_rows // _BEAM
    grid = (n_rows // _ROWS,)
    sc_sel = jnp.take(scores, position - 1, axis=2).reshape(grid[0], _ROWS, 1)
    eos = (jnp.take(generated_tokens, position, axis=1) == _EOS)
    eos = eos.astype(jnp.int32).reshape(grid[0], _ROWS, 1)

    out3 = (grid[0], _ROWS // _BEAM, _BEAM)
    ts, wi, bi = pl.pallas_call(
        _step,
        grid=grid,
        in_specs=[
            pl.BlockSpec((_ROWS, _VOCAB), lambda i: (i, 0)),
            pl.BlockSpec(sc_sel.shape, lambda i: (0, 0, 0)),
            pl.BlockSpec(eos.shape, lambda i: (0, 0, 0)),
        ],
        out_specs=[
            pl.BlockSpec(out3, lambda i: (0, 0, 0)),
            pl.BlockSpec(out3, lambda i: (0, 0, 0)),
            pl.BlockSpec(out3, lambda i: (0, 0, 0)),
        ],
        out_shape=[
            jax.ShapeDtypeStruct(out3, jnp.float32),
            jax.ShapeDtypeStruct(out3, jnp.int32),
            jax.ShapeDtypeStruct(out3, jnp.int32),
        ],
        compiler_params=pltpu.CompilerParams(
            dimension_semantics=("parallel",),
        ),
    )(logits, sc_sel, eos)
    return (ts.reshape(n_batch, _BEAM),
            wi.reshape(n_batch, _BEAM),
            bi.reshape(n_batch, _BEAM))


# speedup vs baseline: 1.1766x; 1.1766x over previous
import jax
import jax.numpy as jnp
from jax.experimental import pallas as pl
from jax.experimental.pallas import tpu as pltpu

_BEAM = 4
_VOCAB = 100000
_EOS = 2
_ROWS = 16
_LANE = 128
_NFULL = _VOCAB // _LANE
_TAILSTART = _VOCAB - _LANE
_TAILSKIP = _NFULL * _LANE - _TAILSTART
_PHASES = 4
_NEG = float("-inf")
_BIG = 2147483647


def _step(x_ref, sc_ref, eos_ref, ts_ref, wi_ref, bi_ref):
    i = pl.program_id(0)
    sc = sc_ref[i]
    eosb = eos_ref[i] != 0

    lane = jax.lax.broadcasted_iota(jnp.int32, (_ROWS, _LANE), 1)
    row8 = jax.lax.broadcasted_iota(jnp.int32, (_ROWS, _LANE), 0)
    row4 = row8 % _BEAM
    flatbase = row4 * _VOCAB + lane

    v0 = x_ref[:, 0:_LANE]
    neg = jnp.full((_ROWS, _LANE), _NEG, jnp.float32)
    zero = jnp.zeros((_ROWS, _LANE), jnp.int32)
    tv = [jnp.where(lane < 2, _NEG, v0), neg, neg, neg]
    ti = [zero, zero, zero, zero]
    for c in range(1, _NFULL):
        off = c * _LANE
        p = (c - 1) % _PHASES
        v = x_ref[:, off:off + _LANE]
        b = v > tv[p]
        tv[p] = jnp.where(b, v, tv[p])
        ti[p] = jnp.where(b, off, ti[p])
    vt = jnp.where(lane >= _TAILSKIP, x_ref[:, _TAILSTART:_TAILSTART + _LANE],
                   _NEG)
    bt = vt > tv[0]
    tv[0] = jnp.where(bt, vt, tv[0])
    ti[0] = jnp.where(bt, _TAILSTART, ti[0])

    mrow = jnp.maximum(jnp.maximum(tv[0], tv[1]), jnp.maximum(tv[2], tv[3]))
    mrow = jnp.maximum(mrow, jnp.where(lane < 2, v0, _NEG))
    m = jnp.max(mrow, axis=1, keepdims=True)

    zerof = jnp.zeros((_ROWS, _LANE), jnp.float32)
    parts = [zerof, zerof, zerof, zerof]
    for c in range(_NFULL):
        off = c * _LANE
        parts[c % _PHASES] = parts[c % _PHASES] + \
            jnp.exp(x_ref[:, off:off + _LANE] - m)
    parts[0] = parts[0] + jnp.exp(vt - m)
    ssum = (parts[0] + parts[1]) + (parts[2] + parts[3])
    s = jnp.sum(ssum, axis=1, keepdims=True)
    crow = sc - m - jnp.log(s)
    crow = jnp.where(eosb, _NEG, crow)

    cv = [t + crow for t in tv] + [jnp.where(eosb & (lane == _EOS), sc, _NEG)]
    cf = [flatbase + i for i in ti] + [flatbase]

    def _treemax(xs):
        while len(xs) > 1:
            xs = [jnp.maximum(a, b) for a, b in zip(xs[::2], xs[1::2])] + \
                ([xs[-1]] if len(xs) % 2 else [])
        return xs[0]

    def _treemin(xs):
        while len(xs) > 1:
            xs = [jnp.minimum(a, b) for a, b in zip(xs[::2], xs[1::2])] + \
                ([xs[-1]] if len(xs) % 2 else [])
        return xs[0]

    kcol = jax.lax.broadcasted_iota(jnp.int32, (1, _ROWS // _BEAM, _BEAM), 2)
    brow = jax.lax.broadcasted_iota(jnp.int32, (1, _ROWS // _BEAM, _BEAM), 1)
    ts = jnp.zeros((1, _ROWS // _BEAM, _BEAM), jnp.float32)
    wi = jnp.zeros((1, _ROWS // _BEAM, _BEAM), jnp.int32)
    bi = jnp.zeros((1, _ROWS // _BEAM, _BEAM), jnp.int32)
    v4s = []
    for b in range(_ROWS // _BEAM):
        selb = (row8 // _BEAM) == b
        av = [jnp.where(selb, v, _NEG) for v in cv]
        for k in range(_BEAM):
            mk = jnp.max(_treemax(av))
            idx = jnp.min(_treemin(
                [jnp.where(v == mk, f, _BIG) for v, f in zip(av, cf)]))
            av = [jnp.where((f == idx) & (v == mk), _NEG, v)
                  for v, f in zip(av, cf)]
            beam = idx // _VOCAB
            word = idx - beam * _VOCAB
            hit = (brow == b) & (kcol == k)
            ts = jnp.where(hit, mk, ts)
            wi = jnp.where(hit, word, wi)
            bi = jnp.where(hit, beam, bi)
            if k == _BEAM - 1:
                v4s.append(mk)

    rio = jax.lax.broadcasted_iota(jnp.int32, (_ROWS, 1), 0)
    v4b = jnp.full((_ROWS, 1), v4s[0])
    for b in range(1, _ROWS // _BEAM):
        v4b = jnp.where(rio >= b * _BEAM, v4s[b], v4b)
    cgt = [zero, zero, zero, zero]
    cge = [zero, zero, zero, zero]

    def _count(p, v):
        a = v + crow
        cgt[p] = cgt[p] + jnp.where(a > v4b, 1, 0)
        cge[p] = cge[p] + jnp.where(a >= v4b, 1, 0)

    _count(0, jnp.where(lane < 2, _NEG, v0))
    for c in range(1, _NFULL):
        off = c * _LANE
        _count((c - 1) % _PHASES, x_ref[:, off:off + _LANE])
    _count(0, vt)
    gts = (cgt[0] + cgt[1]) + (cgt[2] + cgt[3])
    ges = (cge[0] + cge[1]) + (cge[2] + cge[3])
    egt = jnp.where(eosb & (sc > v4b), 1, 0)
    ege = jnp.where(eosb & (sc >= v4b), 1, 0)
    ok = True
    for b in range(_ROWS // _BEAM):
        sel2 = (row8 // _BEAM) == b
        sel1 = (rio // _BEAM) == b
        ngt = jnp.sum(jnp.where(sel2, gts, 0)) + jnp.sum(jnp.where(sel1, egt, 0))
        nge = jnp.sum(jnp.where(sel2, ges, 0)) + jnp.sum(jnp.where(sel1, ege, 0))
        ok = ok & (ngt == 3) & (nge == 4)

    def _good():
        return ts, wi, bi

    def _bad():
        col = jax.lax.broadcasted_iota(jnp.int32, (_ROWS, _VOCAB), 1)
        rowf = jax.lax.broadcasted_iota(jnp.int32, (_ROWS, _VOCAB), 0)
        x = x_ref[...]
        adj = x + crow
        adj = jnp.where(col < 2, _NEG, adj)
        adj = jnp.where((col == _EOS) & eosb, sc, adj)
        fts = jnp.zeros((1, _ROWS // _BEAM, _BEAM), jnp.float32)
        fwi = jnp.zeros((1, _ROWS // _BEAM, _BEAM), jnp.int32)
        fbi = jnp.zeros((1, _ROWS // _BEAM, _BEAM), jnp.int32)
        for b in range(_ROWS // _BEAM):
            ab = adj[b * _BEAM:(b + 1) * _BEAM]
            fl = (rowf[b * _BEAM:(b + 1) * _BEAM] - b * _BEAM) * _VOCAB + \
                col[b * _BEAM:(b + 1) * _BEAM]
            for k in range(_BEAM):
                mk = jnp.max(ab)
                idx = jnp.min(jnp.where(ab == mk, fl, _BIG))
                ab = jnp.where(fl == idx, _NEG, ab)
                beam = idx // _VOCAB
                word = idx - beam * _VOCAB
                hit = (brow == b) & (kcol == k)
                fts = jnp.where(hit, mk, fts)
                fwi = jnp.where(hit, word, fwi)
                fbi = jnp.where(hit, beam, fbi)
        return fts, fwi, fbi

    ts, wi, bi = jax.lax.cond(ok, _good, _bad)
    ts_ref[pl.ds(i, 1)] = ts
    wi_ref[pl.ds(i, 1)] = wi
    bi_ref[pl.ds(i, 1)] = bi


def kernel(logits, scores, generated_tokens, position):
    n_rows = logits.shape[0]
    n_batch = n_rows // _BEAM
    grid = (n_rows // _ROWS,)
    sc_sel = jnp.take(scores, position - 1, axis=2).reshape(grid[0], _ROWS, 1)
    eos = (jnp.take(generated_tokens, position, axis=1) == _EOS)
    eos = eos.astype(jnp.int32).reshape(grid[0], _ROWS, 1)

    out3 = (grid[0], _ROWS // _BEAM, _BEAM)
    ts, wi, bi = pl.pallas_call(
        _step,
        grid=grid,
        in_specs=[
            pl.BlockSpec((_ROWS, _VOCAB), lambda i: (i, 0)),
            pl.BlockSpec(sc_sel.shape, lambda i: (0, 0, 0)),
            pl.BlockSpec(eos.shape, lambda i: (0, 0, 0)),
        ],
        out_specs=[
            pl.BlockSpec(out3, lambda i: (0, 0, 0)),
            pl.BlockSpec(out3, lambda i: (0, 0, 0)),
            pl.BlockSpec(out3, lambda i: (0, 0, 0)),
        ],
        out_shape=[
            jax.ShapeDtypeStruct(out3, jnp.float32),
            jax.ShapeDtypeStruct(out3, jnp.int32),
            jax.ShapeDtypeStruct(out3, jnp.int32),
        ],
        compiler_params=pltpu.CompilerParams(
            dimension_semantics=("parallel",),
        ),
    )(logits, sc_sel, eos)
    return (ts.reshape(n_batch, _BEAM),
            wi.reshape(n_batch, _BEAM),
            bi.reshape(n_batch, _BEAM))
